# Initial kernel scaffold; baseline (speedup 1.0000x reference)
import jax
import jax.numpy as jnp
from jax.experimental import pallas as pl


def kernel(x, edge_index, edge_attr, edge_len_emb, W1, W2, w_node):
    raise NotImplementedError("write your pallas kernel here")


# speedup vs baseline: 1.9360x; 1.9360x over previous
import functools
import math

import jax
import jax.numpy as jnp
from jax import lax
from jax.experimental import pallas as pl
from jax.experimental.pallas import tpu as pltpu
from jax.experimental.pallas import tpu_sc as plsc

N = 10000
E = 160000
DN = 16
DA = 4
DO = 4
DP = 16
EMB = 16
H = 64

NC, NS = 2, 16
NW = NC * NS
EPW = E // NW
CH = 128
NFULL = EPW // CH
TAIL = EPW - NFULL * CH
NDW = 10
RPS = N // NDW

@functools.lru_cache(maxsize=None)
def _sc_kernels():
    mesh = plsc.VectorSubcoreMesh(
        core_axis_name="c", subcore_axis_name="s", num_cores=NC, num_subcores=NS
    )

    @functools.partial(
        pl.kernel,
        out_type=(
            jax.ShapeDtypeStruct((E, DN), jnp.float32),
            jax.ShapeDtypeStruct((E, DN), jnp.float32),
        ),
        mesh=mesh,
        scratch_types=[
            pltpu.VMEM((CH,), jnp.int32),
            pltpu.VMEM((TAIL,), jnp.int32),
            pltpu.VMEM((CH, DN), jnp.float32),
            pltpu.VMEM((TAIL, DN), jnp.float32),
            pltpu.SemaphoreType.DMA,
        ],
        compiler_params=pltpu.CompilerParams(use_tc_tiling_on_sc=False),
    )
    def _sc_gather(x_hbm, ii_hbm, jj_hbm, xi_hbm, xj_hbm, idx_v, tidx_v, rows_v,
                   trows_v, sem):
        cid = lax.axis_index("c")
        sid = lax.axis_index("s")
        w = sid * NC + cid
        base = w * EPW
        for src_hbm, out_hbm in ((ii_hbm, xi_hbm), (jj_hbm, xj_hbm)):

            def chunk(c, carry):
                off = base + c * CH
                pltpu.sync_copy(src_hbm.at[pl.ds(off, CH)], idx_v)
                pltpu.async_copy(x_hbm.at[idx_v], rows_v, sem).wait()
                pltpu.sync_copy(rows_v, out_hbm.at[pl.ds(off, CH)])
                return carry

            lax.fori_loop(0, NFULL, chunk, 0)
            toff = base + NFULL * CH
            pltpu.sync_copy(src_hbm.at[pl.ds(toff, TAIL)], tidx_v)
            pltpu.async_copy(x_hbm.at[tidx_v], trows_v, sem).wait()
            pltpu.sync_copy(trows_v, out_hbm.at[pl.ds(toff, TAIL)])

    @functools.partial(
        pl.kernel,
        out_type=jax.ShapeDtypeStruct((NC, N, DP), jnp.float32),
        mesh=mesh,
        scratch_types=[
            pltpu.VMEM((CH,), jnp.int32),
            pltpu.VMEM((TAIL,), jnp.int32),
            pltpu.VMEM((CH, DP), jnp.float32),
            pltpu.VMEM((TAIL, DP), jnp.float32),
            pltpu.VMEM_SHARED((N, DP), jnp.float32),
            pltpu.SemaphoreType.DMA,
        ],
        compiler_params=pltpu.CompilerParams(use_tc_tiling_on_sc=False),
    )
    def _sc_scatter(eo_hbm, jj_hbm, zer_hbm, out_hbm, idx_v, tidx_v, rows_v,
                    trows_v, shared, sem):
        cid = lax.axis_index("c")
        sid = lax.axis_index("s")
        w = sid * NC + cid
        base = w * EPW

        @pl.when(sid == 0)
        def _():
            pltpu.sync_copy(zer_hbm, shared)

        plsc.subcore_barrier()

        def chunk(c, carry):
            off = base + c * CH
            pltpu.sync_copy(jj_hbm.at[pl.ds(off, CH)], idx_v)
            pltpu.sync_copy(eo_hbm.at[pl.ds(off, CH)], rows_v)
            pltpu.sync_copy(rows_v, shared.at[idx_v], add=True)
            return carry

        lax.fori_loop(0, NFULL, chunk, 0)
        toff = base + NFULL * CH
        pltpu.sync_copy(jj_hbm.at[pl.ds(toff, TAIL)], tidx_v)
        pltpu.sync_copy(eo_hbm.at[pl.ds(toff, TAIL)], trows_v)
        pltpu.sync_copy(trows_v, shared.at[tidx_v], add=True)

        plsc.subcore_barrier()

        @pl.when(sid < NDW)
        def _():
            pltpu.sync_copy(
                shared.at[pl.ds(sid * RPS, RPS)],
                out_hbm.at[cid, pl.ds(sid * RPS, RPS)],
            )

    return _sc_gather, _sc_scatter


BE = 4000


def _tc_edge_body(emb_ref, attr_ref, xi_ref, xj_ref, w1_ref, w2_ref, out_ref):
    h = jnp.dot(emb_ref[...], w1_ref[...], preferred_element_type=jnp.float32,
                precision=lax.Precision.HIGHEST)
    h = h * jax.nn.sigmoid(h)
    a = attr_ref[...]
    q = jnp.concatenate([a[:, i:i + 1] * h for i in range(DA)], axis=1)
    bq = jnp.dot(q, w2_ref[...], preferred_element_type=jnp.float32,
                 precision=lax.Precision.HIGHEST)
    z = jnp.concatenate([xi_ref[...], xj_ref[...]], axis=1)
    cols = [
        jnp.sum(bq[:, k * (2 * DN):(k + 1) * (2 * DN)] * z, axis=1, keepdims=True)
        for k in range(DO)
    ]
    pad = jnp.zeros((BE, DP - DO), jnp.float32)
    out_ref[...] = jnp.concatenate(cols + [pad], axis=1)


def _tc_edge(emb, attr, xi, xj, w1s, w2q):
    return pl.pallas_call(
        _tc_edge_body,
        grid=(E // BE,),
        in_specs=[
            pl.BlockSpec((BE, EMB), lambda b: (b, 0)),
            pl.BlockSpec((BE, DA), lambda b: (b, 0)),
            pl.BlockSpec((BE, DN), lambda b: (b, 0)),
            pl.BlockSpec((BE, DN), lambda b: (b, 0)),
            pl.BlockSpec((EMB, H), lambda b: (0, 0)),
            pl.BlockSpec((DA * H, DO * 2 * DN), lambda b: (0, 0)),
        ],
        out_specs=pl.BlockSpec((BE, DP), lambda b: (b, 0)),
        out_shape=jax.ShapeDtypeStruct((E, DP), jnp.float32),
    )(emb, attr, xi, xj, w1s, w2q)


BN = 2000


def _tc_node_body(x_ref, ps_ref, wn_ref, out_ref):
    e = (ps_ref[0] + ps_ref[1])[:, :DO]
    x = x_ref[...]
    acc = (jnp.dot(x, wn_ref[0], preferred_element_type=jnp.float32,
                   precision=lax.Precision.HIGHEST) * e[:, 0:1])
    for j in range(1, DO):
        acc += (jnp.dot(x, wn_ref[j], preferred_element_type=jnp.float32,
                        precision=lax.Precision.HIGHEST) * e[:, j:j + 1])
    out_ref[...] = acc


def _tc_node(x, ps, wn):
    return pl.pallas_call(
        _tc_node_body,
        grid=(N // BN,),
        in_specs=[
            pl.BlockSpec((BN, DN), lambda b: (b, 0)),
            pl.BlockSpec((NC, BN, DP), lambda b: (0, b, 0)),
            pl.BlockSpec((DO, DN, DN), lambda b: (0, 0, 0)),
        ],
        out_specs=pl.BlockSpec((BN, DN), lambda b: (b, 0)),
        out_shape=jax.ShapeDtypeStruct((N, DN), jnp.float32),
    )(x, ps, wn)


def kernel(x, edge_index, edge_attr, edge_len_emb, W1, W2, w_node):
    ii = edge_index[0]
    jj = edge_index[1]
    w1s = W1 * (1.0 / math.sqrt(EMB))
    w2q = (
        W2.reshape(H, DA, 2 * DN, DO).transpose(1, 0, 3, 2).reshape(DA * H, DO * 2 * DN)
        * (1.0 / (math.sqrt(H) * math.sqrt(DA * 2 * DN)))
    )
    wn = w_node.transpose(1, 0, 2) * (1.0 / math.sqrt(DN * DO))
    zer = jnp.zeros((N, DP), jnp.float32)

    sc_gather, sc_scatter = _sc_kernels()
    xi, xj = sc_gather(x, ii, jj)
    eop = _tc_edge(edge_len_emb, edge_attr, xi, xj, w1s, w2q)
    ps = sc_scatter(eop, jj, zer)
    xo = _tc_node(x, ps, wn)
    return (xo, eop[:, :DO])


# speedup vs baseline: 2.2656x; 1.1702x over previous
import functools
import math

import jax
import jax.numpy as jnp
from jax import lax
from jax.experimental import pallas as pl
from jax.experimental.pallas import tpu as pltpu
from jax.experimental.pallas import tpu_sc as plsc

N = 10000
E = 160000
DN = 16
DA = 4
DO = 4
DP = 16
EMB = 16
H = 64

NC, NS = 2, 16
NW = NC * NS
EPW = E // NW
CH = 128
NFULL = EPW // CH
TAIL = EPW - NFULL * CH
NDW = 10
RPS = N // NDW

@functools.lru_cache(maxsize=None)
def _sc_kernels():
    mesh = plsc.VectorSubcoreMesh(
        core_axis_name="c", subcore_axis_name="s", num_cores=NC, num_subcores=NS
    )

    @functools.partial(
        pl.kernel,
        out_type=(
            jax.ShapeDtypeStruct((E, DN), jnp.float32),
            jax.ShapeDtypeStruct((E, DN), jnp.float32),
        ),
        mesh=mesh,
        scratch_types=[
            pltpu.VMEM((CH,), jnp.int32),
            pltpu.VMEM((TAIL,), jnp.int32),
            pltpu.VMEM((CH, DN), jnp.float32),
            pltpu.VMEM((TAIL, DN), jnp.float32),
            pltpu.SemaphoreType.DMA,
        ],
        compiler_params=pltpu.CompilerParams(use_tc_tiling_on_sc=False),
    )
    def _sc_gather(x_hbm, ii_hbm, jj_hbm, xi_hbm, xj_hbm, idx_v, tidx_v, rows_v,
                   trows_v, sem):
        cid = lax.axis_index("c")
        sid = lax.axis_index("s")
        w = sid * NC + cid
        base = w * EPW
        for src_hbm, out_hbm in ((ii_hbm, xi_hbm), (jj_hbm, xj_hbm)):

            def chunk(c, carry):
                off = base + c * CH
                pltpu.sync_copy(src_hbm.at[pl.ds(off, CH)], idx_v)
                pltpu.async_copy(x_hbm.at[idx_v], rows_v, sem).wait()
                pltpu.sync_copy(rows_v, out_hbm.at[pl.ds(off, CH)])
                return carry

            lax.fori_loop(0, NFULL, chunk, 0)
            toff = base + NFULL * CH
            pltpu.sync_copy(src_hbm.at[pl.ds(toff, TAIL)], tidx_v)
            pltpu.async_copy(x_hbm.at[tidx_v], trows_v, sem).wait()
            pltpu.sync_copy(trows_v, out_hbm.at[pl.ds(toff, TAIL)])

    @functools.partial(
        pl.kernel,
        out_type=jax.ShapeDtypeStruct((NC, N, DP), jnp.float32),
        mesh=mesh,
        scratch_types=[
            pltpu.VMEM((CH,), jnp.int32),
            pltpu.VMEM((TAIL,), jnp.int32),
            pltpu.VMEM((CH, DP), jnp.float32),
            pltpu.VMEM((TAIL, DP), jnp.float32),
            pltpu.VMEM_SHARED((N, DP), jnp.float32),
            pltpu.SemaphoreType.DMA,
        ],
        compiler_params=pltpu.CompilerParams(use_tc_tiling_on_sc=False),
    )
    def _sc_scatter(eo_hbm, jj_hbm, zer_hbm, out_hbm, idx_v, tidx_v, rows_v,
                    trows_v, shared, sem):
        cid = lax.axis_index("c")
        sid = lax.axis_index("s")
        w = sid * NC + cid
        base = w * EPW

        @pl.when(sid == 0)
        def _():
            pltpu.sync_copy(zer_hbm, shared)

        plsc.subcore_barrier()

        def chunk(c, carry):
            off = base + c * CH
            pltpu.sync_copy(jj_hbm.at[pl.ds(off, CH)], idx_v)
            pltpu.sync_copy(eo_hbm.at[pl.ds(off, CH)], rows_v)
            pltpu.sync_copy(rows_v, shared.at[idx_v], add=True)
            return carry

        lax.fori_loop(0, NFULL, chunk, 0)
        toff = base + NFULL * CH
        pltpu.sync_copy(jj_hbm.at[pl.ds(toff, TAIL)], tidx_v)
        pltpu.sync_copy(eo_hbm.at[pl.ds(toff, TAIL)], trows_v)
        pltpu.sync_copy(trows_v, shared.at[tidx_v], add=True)

        plsc.subcore_barrier()

        @pl.when(sid < NDW)
        def _():
            pltpu.sync_copy(
                shared.at[pl.ds(sid * RPS, RPS)],
                out_hbm.at[cid, pl.ds(sid * RPS, RPS)],
            )

    return _sc_gather, _sc_scatter


BE = 4000


def _tc_edge_body(emb_ref, attr_ref, xi_ref, xj_ref, w1_ref, w2_ref, m_ref,
                  out_ref):
    h = jnp.dot(emb_ref[...], w1_ref[...], preferred_element_type=jnp.float32,
                precision=lax.Precision.HIGHEST)
    h = h * jax.nn.sigmoid(h)
    c = jnp.dot(h, w2_ref[...], preferred_element_type=jnp.float32,
                precision=lax.Precision.HIGHEST)
    a = attr_ref[...]
    L = DO * 2 * DN
    bq = a[:, 0:1] * c[:, 0:L]
    for i in range(1, DA):
        bq += a[:, i:i + 1] * c[:, i * L:(i + 1) * L]
    z = jnp.concatenate([xi_ref[...], xj_ref[...]], axis=1)
    z4 = jnp.concatenate([z] * DO, axis=1)
    out_ref[...] = jnp.dot(bq * z4, m_ref[...],
                           preferred_element_type=jnp.float32,
                           precision=lax.Precision.HIGHEST)


def _tc_edge(emb, attr, xi, xj, w1s, w2c, msel):
    return pl.pallas_call(
        _tc_edge_body,
        grid=(E // BE,),
        in_specs=[
            pl.BlockSpec((BE, EMB), lambda b: (b, 0)),
            pl.BlockSpec((BE, DA), lambda b: (b, 0)),
            pl.BlockSpec((BE, DN), lambda b: (b, 0)),
            pl.BlockSpec((BE, DN), lambda b: (b, 0)),
            pl.BlockSpec((EMB, H), lambda b: (0, 0)),
            pl.BlockSpec((H, DA * DO * 2 * DN), lambda b: (0, 0)),
            pl.BlockSpec((DO * 2 * DN, DP), lambda b: (0, 0)),
        ],
        out_specs=pl.BlockSpec((BE, DP), lambda b: (b, 0)),
        out_shape=jax.ShapeDtypeStruct((E, DP), jnp.float32),
    )(emb, attr, xi, xj, w1s, w2c, msel)


BN = 2000


def _tc_node_body(x_ref, ps_ref, wn_ref, out_ref):
    e = (ps_ref[0] + ps_ref[1])[:, :DO]
    x = x_ref[...]
    acc = (jnp.dot(x, wn_ref[0], preferred_element_type=jnp.float32,
                   precision=lax.Precision.HIGHEST) * e[:, 0:1])
    for j in range(1, DO):
        acc += (jnp.dot(x, wn_ref[j], preferred_element_type=jnp.float32,
                        precision=lax.Precision.HIGHEST) * e[:, j:j + 1])
    out_ref[...] = acc


def _tc_node(x, ps, wn):
    return pl.pallas_call(
        _tc_node_body,
        grid=(N // BN,),
        in_specs=[
            pl.BlockSpec((BN, DN), lambda b: (b, 0)),
            pl.BlockSpec((NC, BN, DP), lambda b: (0, b, 0)),
            pl.BlockSpec((DO, DN, DN), lambda b: (0, 0, 0)),
        ],
        out_specs=pl.BlockSpec((BN, DN), lambda b: (b, 0)),
        out_shape=jax.ShapeDtypeStruct((N, DN), jnp.float32),
    )(x, ps, wn)


def kernel(x, edge_index, edge_attr, edge_len_emb, W1, W2, w_node):
    ii = edge_index[0]
    jj = edge_index[1]
    w1s = W1 * (1.0 / math.sqrt(EMB))
    w2c = (
        W2.reshape(H, DA, 2 * DN, DO).transpose(0, 1, 3, 2).reshape(H, DA * DO * 2 * DN)
        * (1.0 / (math.sqrt(H) * math.sqrt(DA * 2 * DN)))
    )
    rows = jnp.arange(DO * 2 * DN)[:, None] // (2 * DN)
    cols = jnp.arange(DP)[None, :]
    msel = (rows == cols).astype(jnp.float32)
    wn = w_node.transpose(1, 0, 2) * (1.0 / math.sqrt(DN * DO))
    zer = jnp.zeros((N, DP), jnp.float32)

    sc_gather, sc_scatter = _sc_kernels()
    xi, xj = sc_gather(x, ii, jj)
    eop = _tc_edge(edge_len_emb, edge_attr, xi, xj, w1s, w2c, msel)
    ps = sc_scatter(eop, jj, zer)
    xo = _tc_node(x, ps, wn)
    return (xo, eop[:, :DO])


# speedup vs baseline: 3.1237x; 1.3788x over previous
import functools
import math

import jax
import jax.numpy as jnp
from jax import lax
from jax.experimental import pallas as pl
from jax.experimental.pallas import tpu as pltpu
from jax.experimental.pallas import tpu_sc as plsc

N = 10000
E = 160000
DN = 16
DA = 4
DO = 4
DP = 16
EMB = 16
H = 64

NC, NS = 2, 16
NW = NC * NS
EPW = E // NW
CH = 128
NFULL = EPW // CH
TAIL = EPW - NFULL * CH
NDW = 10
RPS = N // NDW

@functools.lru_cache(maxsize=None)
def _sc_kernels():
    mesh = plsc.VectorSubcoreMesh(
        core_axis_name="c", subcore_axis_name="s", num_cores=NC, num_subcores=NS
    )

    @functools.partial(
        pl.kernel,
        out_type=(
            jax.ShapeDtypeStruct((E, DN), jnp.float32),
            jax.ShapeDtypeStruct((E, DN), jnp.float32),
        ),
        mesh=mesh,
        scratch_types=[
            pltpu.VMEM((CH,), jnp.int32),
            pltpu.VMEM((TAIL,), jnp.int32),
            pltpu.VMEM((CH, DN), jnp.float32),
            pltpu.VMEM((TAIL, DN), jnp.float32),
            pltpu.SemaphoreType.DMA,
        ],
        compiler_params=pltpu.CompilerParams(use_tc_tiling_on_sc=False),
    )
    def _sc_gather(x_hbm, ii_hbm, jj_hbm, xi_hbm, xj_hbm, idx_v, tidx_v, rows_v,
                   trows_v, sem):
        cid = lax.axis_index("c")
        sid = lax.axis_index("s")
        w = sid * NC + cid
        base = w * EPW
        for src_hbm, out_hbm in ((ii_hbm, xi_hbm), (jj_hbm, xj_hbm)):

            def chunk(c, carry):
                off = base + c * CH
                pltpu.sync_copy(src_hbm.at[pl.ds(off, CH)], idx_v)
                pltpu.async_copy(x_hbm.at[idx_v], rows_v, sem).wait()
                pltpu.sync_copy(rows_v, out_hbm.at[pl.ds(off, CH)])
                return carry

            lax.fori_loop(0, NFULL, chunk, 0)
            toff = base + NFULL * CH
            pltpu.sync_copy(src_hbm.at[pl.ds(toff, TAIL)], tidx_v)
            pltpu.async_copy(x_hbm.at[tidx_v], trows_v, sem).wait()
            pltpu.sync_copy(trows_v, out_hbm.at[pl.ds(toff, TAIL)])

    @functools.partial(
        pl.kernel,
        out_type=jax.ShapeDtypeStruct((NC, N, DP), jnp.float32),
        mesh=mesh,
        scratch_types=[
            pltpu.VMEM((CH,), jnp.int32),
            pltpu.VMEM((TAIL,), jnp.int32),
            pltpu.VMEM((CH, DP), jnp.float32),
            pltpu.VMEM((TAIL, DP), jnp.float32),
            pltpu.VMEM_SHARED((N, DP), jnp.float32),
            pltpu.SemaphoreType.DMA,
        ],
        compiler_params=pltpu.CompilerParams(use_tc_tiling_on_sc=False),
    )
    def _sc_scatter(eo_hbm, jj_hbm, zer_hbm, out_hbm, idx_v, tidx_v, rows_v,
                    trows_v, shared, sem):
        cid = lax.axis_index("c")
        sid = lax.axis_index("s")
        w = sid * NC + cid
        base = w * EPW

        @pl.when(sid == 0)
        def _():
            pltpu.sync_copy(zer_hbm, shared)

        plsc.subcore_barrier()

        def chunk(c, carry):
            off = base + c * CH
            pltpu.sync_copy(jj_hbm.at[pl.ds(off, CH)], idx_v)
            pltpu.sync_copy(eo_hbm.at[pl.ds(off, CH)], rows_v)
            pltpu.sync_copy(rows_v, shared.at[idx_v], add=True)
            return carry

        lax.fori_loop(0, NFULL, chunk, 0)
        toff = base + NFULL * CH
        pltpu.sync_copy(jj_hbm.at[pl.ds(toff, TAIL)], tidx_v)
        pltpu.sync_copy(eo_hbm.at[pl.ds(toff, TAIL)], trows_v)
        pltpu.sync_copy(trows_v, shared.at[tidx_v], add=True)

        plsc.subcore_barrier()

        @pl.when(sid < NDW)
        def _():
            pltpu.sync_copy(
                shared.at[pl.ds(sid * RPS, RPS)],
                out_hbm.at[cid, pl.ds(sid * RPS, RPS)],
            )

    return _sc_gather, _sc_scatter


BE = 4000


def _tc_edge_body(emb_ref, attr_ref, xi_ref, xj_ref, w1_ref, w2_ref, m_ref,
                  out_ref):
    h = jnp.dot(emb_ref[...], w1_ref[...], preferred_element_type=jnp.float32)
    h = h * jax.nn.sigmoid(h)
    c = jnp.dot(h, w2_ref[...], preferred_element_type=jnp.float32)
    a = attr_ref[...]
    L = DO * 2 * DN
    bq = a[:, 0:1] * c[:, 0:L]
    for i in range(1, DA):
        bq += a[:, i:i + 1] * c[:, i * L:(i + 1) * L]
    z = jnp.concatenate([xi_ref[...], xj_ref[...]], axis=1)
    z4 = jnp.concatenate([z] * DO, axis=1)
    out_ref[...] = jnp.dot(bq * z4, m_ref[...],
                           preferred_element_type=jnp.float32,
                           precision=lax.Precision.HIGHEST)


def _tc_edge(emb, attr, xi, xj, w1s, w2c, msel):
    return pl.pallas_call(
        _tc_edge_body,
        grid=(E // BE,),
        in_specs=[
            pl.BlockSpec((BE, EMB), lambda b: (b, 0)),
            pl.BlockSpec((BE, DA), lambda b: (b, 0)),
            pl.BlockSpec((BE, DN), lambda b: (b, 0)),
            pl.BlockSpec((BE, DN), lambda b: (b, 0)),
            pl.BlockSpec((EMB, H), lambda b: (0, 0)),
            pl.BlockSpec((H, DA * DO * 2 * DN), lambda b: (0, 0)),
            pl.BlockSpec((DO * 2 * DN, DP), lambda b: (0, 0)),
        ],
        out_specs=pl.BlockSpec((BE, DP), lambda b: (b, 0)),
        out_shape=jax.ShapeDtypeStruct((E, DP), jnp.float32),
    )(emb, attr, xi, xj, w1s, w2c, msel)


BN = 2000


def _tc_node_body(x_ref, ps_ref, wn_ref, out_ref):
    e = (ps_ref[0] + ps_ref[1])[:, :DO]
    x = x_ref[...]
    acc = (jnp.dot(x, wn_ref[0], preferred_element_type=jnp.float32,
                   precision=lax.Precision.HIGHEST) * e[:, 0:1])
    for j in range(1, DO):
        acc += (jnp.dot(x, wn_ref[j], preferred_element_type=jnp.float32,
                        precision=lax.Precision.HIGHEST) * e[:, j:j + 1])
    out_ref[...] = acc


def _tc_node(x, ps, wn):
    return pl.pallas_call(
        _tc_node_body,
        grid=(N // BN,),
        in_specs=[
            pl.BlockSpec((BN, DN), lambda b: (b, 0)),
            pl.BlockSpec((NC, BN, DP), lambda b: (0, b, 0)),
            pl.BlockSpec((DO, DN, DN), lambda b: (0, 0, 0)),
        ],
        out_specs=pl.BlockSpec((BN, DN), lambda b: (b, 0)),
        out_shape=jax.ShapeDtypeStruct((N, DN), jnp.float32),
    )(x, ps, wn)


def kernel(x, edge_index, edge_attr, edge_len_emb, W1, W2, w_node):
    ii = edge_index[0]
    jj = edge_index[1]
    w1s = W1 * (1.0 / math.sqrt(EMB))
    w2c = (
        W2.reshape(H, DA, 2 * DN, DO).transpose(0, 1, 3, 2).reshape(H, DA * DO * 2 * DN)
        * (1.0 / (math.sqrt(H) * math.sqrt(DA * 2 * DN)))
    )
    rows = jnp.arange(DO * 2 * DN)[:, None] // (2 * DN)
    cols = jnp.arange(DP)[None, :]
    msel = (rows == cols).astype(jnp.float32)
    wn = w_node.transpose(1, 0, 2) * (1.0 / math.sqrt(DN * DO))
    zer = jnp.zeros((N, DP), jnp.float32)

    sc_gather, sc_scatter = _sc_kernels()
    xi, xj = sc_gather(x, ii, jj)
    eop = _tc_edge(edge_len_emb, edge_attr, xi, xj, w1s, w2c, msel)
    ps = sc_scatter(eop, jj, zer)
    xo = _tc_node(x, ps, wn)
    return (xo, eop[:, :DO])


# speedup vs baseline: 3.2571x; 1.0427x over previous
import functools
import math

import jax
import jax.numpy as jnp
from jax import lax
from jax.experimental import pallas as pl
from jax.experimental.pallas import tpu as pltpu
from jax.experimental.pallas import tpu_sc as plsc

N = 10000
E = 160000
DN = 16
DA = 4
DO = 4
DP = 16
EMB = 16
H = 64

NC, NS = 2, 16
NW = NC * NS
EPW = E // NW
CH = 128
NFULL = EPW // CH
TAIL = EPW - NFULL * CH
NDW = 10
RPS = N // NDW

@functools.lru_cache(maxsize=None)
def _sc_kernels():
    mesh = plsc.VectorSubcoreMesh(
        core_axis_name="c", subcore_axis_name="s", num_cores=NC, num_subcores=NS
    )

    @functools.partial(
        pl.kernel,
        out_type=(
            jax.ShapeDtypeStruct((E, DN), jnp.float32),
            jax.ShapeDtypeStruct((E, DN), jnp.float32),
        ),
        mesh=mesh,
        scratch_types=[
            pltpu.VMEM((CH,), jnp.int32),
            pltpu.VMEM((CH,), jnp.int32),
            pltpu.VMEM((TAIL,), jnp.int32),
            pltpu.VMEM((TAIL,), jnp.int32),
            pltpu.VMEM((CH, DN), jnp.float32),
            pltpu.VMEM((CH, DN), jnp.float32),
            pltpu.VMEM((TAIL, DN), jnp.float32),
            pltpu.VMEM((TAIL, DN), jnp.float32),
            pltpu.SemaphoreType.DMA,
            pltpu.SemaphoreType.DMA,
        ],
        compiler_params=pltpu.CompilerParams(use_tc_tiling_on_sc=False),
    )
    def _sc_gather(x_hbm, ii_hbm, jj_hbm, xi_hbm, xj_hbm, idxi_v, idxj_v,
                   tidxi_v, tidxj_v, rowsi_v, rowsj_v, trowsi_v, trowsj_v,
                   semi, semj):
        cid = lax.axis_index("c")
        sid = lax.axis_index("s")
        w = sid * NC + cid
        base = w * EPW

        def chunk(c, carry):
            off = base + c * CH
            pltpu.sync_copy(ii_hbm.at[pl.ds(off, CH)], idxi_v)
            pltpu.sync_copy(jj_hbm.at[pl.ds(off, CH)], idxj_v)
            cpi = pltpu.async_copy(x_hbm.at[idxi_v], rowsi_v, semi)
            cpj = pltpu.async_copy(x_hbm.at[idxj_v], rowsj_v, semj)
            cpi.wait()
            pltpu.sync_copy(rowsi_v, xi_hbm.at[pl.ds(off, CH)])
            cpj.wait()
            pltpu.sync_copy(rowsj_v, xj_hbm.at[pl.ds(off, CH)])
            return carry

        lax.fori_loop(0, NFULL, chunk, 0)
        toff = base + NFULL * CH
        pltpu.sync_copy(ii_hbm.at[pl.ds(toff, TAIL)], tidxi_v)
        pltpu.sync_copy(jj_hbm.at[pl.ds(toff, TAIL)], tidxj_v)
        cpi = pltpu.async_copy(x_hbm.at[tidxi_v], trowsi_v, semi)
        cpj = pltpu.async_copy(x_hbm.at[tidxj_v], trowsj_v, semj)
        cpi.wait()
        pltpu.sync_copy(trowsi_v, xi_hbm.at[pl.ds(toff, TAIL)])
        cpj.wait()
        pltpu.sync_copy(trowsj_v, xj_hbm.at[pl.ds(toff, TAIL)])

    @functools.partial(
        pl.kernel,
        out_type=jax.ShapeDtypeStruct((NC, N, DP), jnp.float32),
        mesh=mesh,
        scratch_types=[
            pltpu.VMEM((CH,), jnp.int32),
            pltpu.VMEM((TAIL,), jnp.int32),
            pltpu.VMEM((CH, DP), jnp.float32),
            pltpu.VMEM((TAIL, DP), jnp.float32),
            pltpu.VMEM_SHARED((N, DP), jnp.float32),
            pltpu.SemaphoreType.DMA,
        ],
        compiler_params=pltpu.CompilerParams(use_tc_tiling_on_sc=False),
    )
    def _sc_scatter(eo_hbm, jj_hbm, zer_hbm, out_hbm, idx_v, tidx_v, rows_v,
                    trows_v, shared, sem):
        cid = lax.axis_index("c")
        sid = lax.axis_index("s")
        w = sid * NC + cid
        base = w * EPW

        @pl.when(sid == 0)
        def _():
            pltpu.sync_copy(zer_hbm, shared)

        plsc.subcore_barrier()

        def chunk(c, carry):
            off = base + c * CH
            pltpu.sync_copy(jj_hbm.at[pl.ds(off, CH)], idx_v)
            pltpu.sync_copy(eo_hbm.at[pl.ds(off, CH)], rows_v)
            pltpu.sync_copy(rows_v, shared.at[idx_v], add=True)
            return carry

        lax.fori_loop(0, NFULL, chunk, 0)
        toff = base + NFULL * CH
        pltpu.sync_copy(jj_hbm.at[pl.ds(toff, TAIL)], tidx_v)
        pltpu.sync_copy(eo_hbm.at[pl.ds(toff, TAIL)], trows_v)
        pltpu.sync_copy(trows_v, shared.at[tidx_v], add=True)

        plsc.subcore_barrier()

        @pl.when(sid < NDW)
        def _():
            pltpu.sync_copy(
                shared.at[pl.ds(sid * RPS, RPS)],
                out_hbm.at[cid, pl.ds(sid * RPS, RPS)],
            )

    return _sc_gather, _sc_scatter


BE = 4000


def _tc_edge_body(emb_ref, attr_ref, xi_ref, xj_ref, w1_ref, w2_ref, m_ref,
                  out_ref, out4_ref):
    h = jnp.dot(emb_ref[...], w1_ref[...], preferred_element_type=jnp.float32)
    h = h * jax.nn.sigmoid(h)
    c = jnp.dot(h, w2_ref[...], preferred_element_type=jnp.float32)
    a = attr_ref[...]
    L = DO * 2 * DN
    bq = a[:, 0:1] * c[:, 0:L]
    for i in range(1, DA):
        bq += a[:, i:i + 1] * c[:, i * L:(i + 1) * L]
    z = jnp.concatenate([xi_ref[...], xj_ref[...]], axis=1)
    z4 = jnp.concatenate([z] * DO, axis=1)
    eo = jnp.dot(bq * z4, m_ref[...], preferred_element_type=jnp.float32,
                 precision=lax.Precision.HIGHEST)
    out_ref[...] = eo
    out4_ref[...] = eo[:, :DO]


def _tc_edge(emb, attr, xi, xj, w1s, w2c, msel):
    return pl.pallas_call(
        _tc_edge_body,
        grid=(E // BE,),
        in_specs=[
            pl.BlockSpec((BE, EMB), lambda b: (b, 0)),
            pl.BlockSpec((BE, DA), lambda b: (b, 0)),
            pl.BlockSpec((BE, DN), lambda b: (b, 0)),
            pl.BlockSpec((BE, DN), lambda b: (b, 0)),
            pl.BlockSpec((EMB, H), lambda b: (0, 0)),
            pl.BlockSpec((H, DA * DO * 2 * DN), lambda b: (0, 0)),
            pl.BlockSpec((DO * 2 * DN, DP), lambda b: (0, 0)),
        ],
        out_specs=[
            pl.BlockSpec((BE, DP), lambda b: (b, 0)),
            pl.BlockSpec((BE, DO), lambda b: (b, 0)),
        ],
        out_shape=[
            jax.ShapeDtypeStruct((E, DP), jnp.float32),
            jax.ShapeDtypeStruct((E, DO), jnp.float32),
        ],
    )(emb, attr, xi, xj, w1s, w2c, msel)


BN = 2000


def _tc_node_body(x_ref, ps_ref, wn_ref, out_ref):
    e = (ps_ref[0] + ps_ref[1])[:, :DO]
    x = x_ref[...]
    acc = (jnp.dot(x, wn_ref[0], preferred_element_type=jnp.float32,
                   precision=lax.Precision.HIGHEST) * e[:, 0:1])
    for j in range(1, DO):
        acc += (jnp.dot(x, wn_ref[j], preferred_element_type=jnp.float32,
                        precision=lax.Precision.HIGHEST) * e[:, j:j + 1])
    out_ref[...] = acc


def _tc_node(x, ps, wn):
    return pl.pallas_call(
        _tc_node_body,
        grid=(N // BN,),
        in_specs=[
            pl.BlockSpec((BN, DN), lambda b: (b, 0)),
            pl.BlockSpec((NC, BN, DP), lambda b: (0, b, 0)),
            pl.BlockSpec((DO, DN, DN), lambda b: (0, 0, 0)),
        ],
        out_specs=pl.BlockSpec((BN, DN), lambda b: (b, 0)),
        out_shape=jax.ShapeDtypeStruct((N, DN), jnp.float32),
    )(x, ps, wn)


def kernel(x, edge_index, edge_attr, edge_len_emb, W1, W2, w_node):
    ii = edge_index[0]
    jj = edge_index[1]
    w1s = W1 * (1.0 / math.sqrt(EMB))
    w2c = (
        W2.reshape(H, DA, 2 * DN, DO).transpose(0, 1, 3, 2).reshape(H, DA * DO * 2 * DN)
        * (1.0 / (math.sqrt(H) * math.sqrt(DA * 2 * DN)))
    )
    rows = jnp.arange(DO * 2 * DN)[:, None] // (2 * DN)
    cols = jnp.arange(DP)[None, :]
    msel = (rows == cols).astype(jnp.float32)
    wn = w_node.transpose(1, 0, 2) * (1.0 / math.sqrt(DN * DO))
    zer = jnp.zeros((N, DP), jnp.float32)

    sc_gather, sc_scatter = _sc_kernels()
    xi, xj = sc_gather(x, ii, jj)
    eop, eo4 = _tc_edge(edge_len_emb, edge_attr, xi, xj, w1s, w2c, msel)
    ps = sc_scatter(eop, jj, zer)
    xo = _tc_node(x, ps, wn)
    return (xo, eo4)


# speedup vs baseline: 3.4093x; 1.0468x over previous
import functools
import math

import jax
import jax.numpy as jnp
from jax import lax
from jax.experimental import pallas as pl
from jax.experimental.pallas import tpu as pltpu
from jax.experimental.pallas import tpu_sc as plsc

N = 10000
E = 160000
DN = 16
DA = 4
DO = 4
DP = 16
EMB = 16
H = 64

NC, NS = 2, 16
NW = NC * NS
EPW = E // NW
CH = 128
NFULL = EPW // CH
TAIL = EPW - NFULL * CH
NDW = 10
RPS = N // NDW

@functools.lru_cache(maxsize=None)
def _sc_kernels():
    mesh = plsc.VectorSubcoreMesh(
        core_axis_name="c", subcore_axis_name="s", num_cores=NC, num_subcores=NS
    )

    @functools.partial(
        pl.kernel,
        out_type=(
            jax.ShapeDtypeStruct((E, DN), jnp.float32),
            jax.ShapeDtypeStruct((E, DN), jnp.float32),
        ),
        mesh=mesh,
        scratch_types=[
            pltpu.VMEM((CH,), jnp.int32),
            pltpu.VMEM((CH,), jnp.int32),
            pltpu.VMEM((TAIL,), jnp.int32),
            pltpu.VMEM((TAIL,), jnp.int32),
            pltpu.VMEM((CH, DN), jnp.float32),
            pltpu.VMEM((CH, DN), jnp.float32),
            pltpu.VMEM((TAIL, DN), jnp.float32),
            pltpu.VMEM((TAIL, DN), jnp.float32),
            pltpu.SemaphoreType.DMA,
            pltpu.SemaphoreType.DMA,
        ],
        compiler_params=pltpu.CompilerParams(use_tc_tiling_on_sc=False),
    )
    def _sc_gather(x_hbm, ii_hbm, jj_hbm, xi_hbm, xj_hbm, idxi_v, idxj_v,
                   tidxi_v, tidxj_v, rowsi_v, rowsj_v, trowsi_v, trowsj_v,
                   semi, semj):
        cid = lax.axis_index("c")
        sid = lax.axis_index("s")
        w = sid * NC + cid
        base = w * EPW

        def chunk(c, carry):
            off = base + c * CH
            pltpu.sync_copy(ii_hbm.at[pl.ds(off, CH)], idxi_v)
            pltpu.sync_copy(jj_hbm.at[pl.ds(off, CH)], idxj_v)
            cpi = pltpu.async_copy(x_hbm.at[idxi_v], rowsi_v, semi)
            cpj = pltpu.async_copy(x_hbm.at[idxj_v], rowsj_v, semj)
            cpi.wait()
            pltpu.sync_copy(rowsi_v, xi_hbm.at[pl.ds(off, CH)])
            cpj.wait()
            pltpu.sync_copy(rowsj_v, xj_hbm.at[pl.ds(off, CH)])
            return carry

        lax.fori_loop(0, NFULL, chunk, 0)
        toff = base + NFULL * CH
        pltpu.sync_copy(ii_hbm.at[pl.ds(toff, TAIL)], tidxi_v)
        pltpu.sync_copy(jj_hbm.at[pl.ds(toff, TAIL)], tidxj_v)
        cpi = pltpu.async_copy(x_hbm.at[tidxi_v], trowsi_v, semi)
        cpj = pltpu.async_copy(x_hbm.at[tidxj_v], trowsj_v, semj)
        cpi.wait()
        pltpu.sync_copy(trowsi_v, xi_hbm.at[pl.ds(toff, TAIL)])
        cpj.wait()
        pltpu.sync_copy(trowsj_v, xj_hbm.at[pl.ds(toff, TAIL)])

    @functools.partial(
        pl.kernel,
        out_type=jax.ShapeDtypeStruct((NC, N, DP), jnp.float32),
        mesh=mesh,
        scratch_types=[
            pltpu.VMEM((CH,), jnp.int32),
            pltpu.VMEM((TAIL,), jnp.int32),
            pltpu.VMEM((CH, DP), jnp.float32),
            pltpu.VMEM((TAIL, DP), jnp.float32),
            pltpu.VMEM_SHARED((N, DP), jnp.float32),
            pltpu.SemaphoreType.DMA,
        ],
        compiler_params=pltpu.CompilerParams(use_tc_tiling_on_sc=False),
    )
    def _sc_scatter(eo_hbm, jj_hbm, zer_hbm, out_hbm, idx_v, tidx_v, rows_v,
                    trows_v, shared, sem):
        cid = lax.axis_index("c")
        sid = lax.axis_index("s")
        w = sid * NC + cid
        base = w * EPW

        @pl.when(sid == 0)
        def _():
            pltpu.sync_copy(zer_hbm, shared)

        plsc.subcore_barrier()

        def chunk(c, carry):
            off = base + c * CH
            pltpu.sync_copy(jj_hbm.at[pl.ds(off, CH)], idx_v)
            pltpu.sync_copy(eo_hbm.at[pl.ds(off, CH)], rows_v)
            pltpu.sync_copy(rows_v, shared.at[idx_v], add=True)
            return carry

        lax.fori_loop(0, NFULL, chunk, 0)
        toff = base + NFULL * CH
        pltpu.sync_copy(jj_hbm.at[pl.ds(toff, TAIL)], tidx_v)
        pltpu.sync_copy(eo_hbm.at[pl.ds(toff, TAIL)], trows_v)
        pltpu.sync_copy(trows_v, shared.at[tidx_v], add=True)

        plsc.subcore_barrier()

        @pl.when(sid < NDW)
        def _():
            pltpu.sync_copy(
                shared.at[pl.ds(sid * RPS, RPS)],
                out_hbm.at[cid, pl.ds(sid * RPS, RPS)],
            )

    return _sc_gather, _sc_scatter


BE = 4000


def _tc_edge_a_body(emb_ref, attr_ref, w1_ref, w2_ref, bq_ref):
    h = jnp.dot(emb_ref[...], w1_ref[...], preferred_element_type=jnp.float32)
    h = h * jax.nn.sigmoid(h)
    c = jnp.dot(h, w2_ref[...], preferred_element_type=jnp.float32)
    a = attr_ref[...]
    L = DO * 2 * DN
    bq = a[:, 0:1] * c[:, 0:L]
    for i in range(1, DA):
        bq += a[:, i:i + 1] * c[:, i * L:(i + 1) * L]
    bq_ref[...] = bq


def _tc_edge_a(emb, attr, w1s, w2c):
    return pl.pallas_call(
        _tc_edge_a_body,
        grid=(E // BE,),
        in_specs=[
            pl.BlockSpec((BE, EMB), lambda b: (b, 0)),
            pl.BlockSpec((BE, DA), lambda b: (b, 0)),
            pl.BlockSpec((EMB, H), lambda b: (0, 0)),
            pl.BlockSpec((H, DA * DO * 2 * DN), lambda b: (0, 0)),
        ],
        out_specs=pl.BlockSpec((BE, DO * 2 * DN), lambda b: (b, 0)),
        out_shape=jax.ShapeDtypeStruct((E, DO * 2 * DN), jnp.float32),
    )(emb, attr, w1s, w2c)


def _tc_edge_b_body(bq_ref, xi_ref, xj_ref, m_ref, out_ref, out4_ref):
    z = jnp.concatenate([xi_ref[...], xj_ref[...]], axis=1)
    z4 = jnp.concatenate([z] * DO, axis=1)
    eo = jnp.dot(bq_ref[...] * z4, m_ref[...],
                 preferred_element_type=jnp.float32,
                 precision=lax.Precision.HIGHEST)
    out_ref[...] = eo
    out4_ref[...] = eo[:, :DO]


def _tc_edge_b(bq, xi, xj, msel):
    return pl.pallas_call(
        _tc_edge_b_body,
        grid=(E // BE,),
        in_specs=[
            pl.BlockSpec((BE, DO * 2 * DN), lambda b: (b, 0)),
            pl.BlockSpec((BE, DN), lambda b: (b, 0)),
            pl.BlockSpec((BE, DN), lambda b: (b, 0)),
            pl.BlockSpec((DO * 2 * DN, DP), lambda b: (0, 0)),
        ],
        out_specs=[
            pl.BlockSpec((BE, DP), lambda b: (b, 0)),
            pl.BlockSpec((BE, DO), lambda b: (b, 0)),
        ],
        out_shape=[
            jax.ShapeDtypeStruct((E, DP), jnp.float32),
            jax.ShapeDtypeStruct((E, DO), jnp.float32),
        ],
    )(bq, xi, xj, msel)


BN = 2000


def _tc_node_body(x_ref, ps_ref, wn_ref, out_ref):
    e = (ps_ref[0] + ps_ref[1])[:, :DO]
    x = x_ref[...]
    acc = (jnp.dot(x, wn_ref[0], preferred_element_type=jnp.float32,
                   precision=lax.Precision.HIGHEST) * e[:, 0:1])
    for j in range(1, DO):
        acc += (jnp.dot(x, wn_ref[j], preferred_element_type=jnp.float32,
                        precision=lax.Precision.HIGHEST) * e[:, j:j + 1])
    out_ref[...] = acc


def _tc_node(x, ps, wn):
    return pl.pallas_call(
        _tc_node_body,
        grid=(N // BN,),
        in_specs=[
            pl.BlockSpec((BN, DN), lambda b: (b, 0)),
            pl.BlockSpec((NC, BN, DP), lambda b: (0, b, 0)),
            pl.BlockSpec((DO, DN, DN), lambda b: (0, 0, 0)),
        ],
        out_specs=pl.BlockSpec((BN, DN), lambda b: (b, 0)),
        out_shape=jax.ShapeDtypeStruct((N, DN), jnp.float32),
    )(x, ps, wn)


def kernel(x, edge_index, edge_attr, edge_len_emb, W1, W2, w_node):
    ii = edge_index[0]
    jj = edge_index[1]
    w1s = W1 * (1.0 / math.sqrt(EMB))
    w2c = (
        W2.reshape(H, DA, 2 * DN, DO).transpose(0, 1, 3, 2).reshape(H, DA * DO * 2 * DN)
        * (1.0 / (math.sqrt(H) * math.sqrt(DA * 2 * DN)))
    )
    rows = jnp.arange(DO * 2 * DN)[:, None] // (2 * DN)
    cols = jnp.arange(DP)[None, :]
    msel = (rows == cols).astype(jnp.float32)
    wn = w_node.transpose(1, 0, 2) * (1.0 / math.sqrt(DN * DO))
    zer = jnp.zeros((N, DP), jnp.float32)

    sc_gather, sc_scatter = _sc_kernels()
    xi, xj = sc_gather(x, ii, jj)
    bq = _tc_edge_a(edge_len_emb, edge_attr, w1s, w2c)
    eop, eo4 = _tc_edge_b(bq, xi, xj, msel)
    ps = sc_scatter(eop, jj, zer)
    xo = _tc_node(x, ps, wn)
    return (xo, eo4)


# speedup vs baseline: 3.5629x; 1.0450x over previous
import functools
import math

import jax
import jax.numpy as jnp
from jax import lax
from jax.experimental import pallas as pl
from jax.experimental.pallas import tpu as pltpu
from jax.experimental.pallas import tpu_sc as plsc

N = 10000
E = 160000
DN = 16
DA = 4
DO = 4
DP = 16
EMB = 16
H = 64

NC, NS = 2, 16
NW = NC * NS
EPW = E // NW
CH = 128
NFULL = EPW // CH
TAIL = EPW - NFULL * CH
NDW = 10
RPS = N // NDW

@functools.lru_cache(maxsize=None)
def _sc_kernels():
    mesh = plsc.VectorSubcoreMesh(
        core_axis_name="c", subcore_axis_name="s", num_cores=NC, num_subcores=NS
    )

    @functools.partial(
        pl.kernel,
        out_type=(
            jax.ShapeDtypeStruct((E, DN), jnp.float32),
            jax.ShapeDtypeStruct((E, DN), jnp.float32),
        ),
        mesh=mesh,
        scratch_types=[
            pltpu.VMEM((CH,), jnp.int32),
            pltpu.VMEM((CH,), jnp.int32),
            pltpu.VMEM((TAIL,), jnp.int32),
            pltpu.VMEM((TAIL,), jnp.int32),
            pltpu.VMEM((CH, DN), jnp.float32),
            pltpu.VMEM((CH, DN), jnp.float32),
            pltpu.VMEM((TAIL, DN), jnp.float32),
            pltpu.VMEM((TAIL, DN), jnp.float32),
            pltpu.SemaphoreType.DMA,
            pltpu.SemaphoreType.DMA,
        ],
        compiler_params=pltpu.CompilerParams(use_tc_tiling_on_sc=False),
    )
    def _sc_gather(x_hbm, ii_hbm, jj_hbm, xi_hbm, xj_hbm, idxi_v, idxj_v,
                   tidxi_v, tidxj_v, rowsi_v, rowsj_v, trowsi_v, trowsj_v,
                   semi, semj):
        cid = lax.axis_index("c")
        sid = lax.axis_index("s")
        w = sid * NC + cid
        base = w * EPW

        def chunk(c, carry):
            off = base + c * CH
            pltpu.sync_copy(ii_hbm.at[pl.ds(off, CH)], idxi_v)
            pltpu.sync_copy(jj_hbm.at[pl.ds(off, CH)], idxj_v)
            cpi = pltpu.async_copy(x_hbm.at[idxi_v], rowsi_v, semi)
            cpj = pltpu.async_copy(x_hbm.at[idxj_v], rowsj_v, semj)
            cpi.wait()
            pltpu.sync_copy(rowsi_v, xi_hbm.at[pl.ds(off, CH)])
            cpj.wait()
            pltpu.sync_copy(rowsj_v, xj_hbm.at[pl.ds(off, CH)])
            return carry

        lax.fori_loop(0, NFULL, chunk, 0)
        toff = base + NFULL * CH
        pltpu.sync_copy(ii_hbm.at[pl.ds(toff, TAIL)], tidxi_v)
        pltpu.sync_copy(jj_hbm.at[pl.ds(toff, TAIL)], tidxj_v)
        cpi = pltpu.async_copy(x_hbm.at[tidxi_v], trowsi_v, semi)
        cpj = pltpu.async_copy(x_hbm.at[tidxj_v], trowsj_v, semj)
        cpi.wait()
        pltpu.sync_copy(trowsi_v, xi_hbm.at[pl.ds(toff, TAIL)])
        cpj.wait()
        pltpu.sync_copy(trowsj_v, xj_hbm.at[pl.ds(toff, TAIL)])

    @functools.partial(
        pl.kernel,
        out_type=jax.ShapeDtypeStruct((NC, N, DP), jnp.float32),
        mesh=mesh,
        scratch_types=[
            pltpu.VMEM((CH,), jnp.int32),
            pltpu.VMEM((TAIL,), jnp.int32),
            pltpu.VMEM((CH, DP), jnp.float32),
            pltpu.VMEM((TAIL, DP), jnp.float32),
            pltpu.VMEM_SHARED((N, DP), jnp.float32),
            pltpu.SemaphoreType.DMA,
        ],
        compiler_params=pltpu.CompilerParams(use_tc_tiling_on_sc=False),
    )
    def _sc_scatter(eo_hbm, jj_hbm, zer_hbm, out_hbm, idx_v, tidx_v, rows_v,
                    trows_v, shared, sem):
        cid = lax.axis_index("c")
        sid = lax.axis_index("s")
        w = sid * NC + cid
        base = w * EPW

        @pl.when(sid == 0)
        def _():
            pltpu.sync_copy(zer_hbm, shared)

        plsc.subcore_barrier()

        def chunk(c, carry):
            off = base + c * CH
            pltpu.sync_copy(jj_hbm.at[pl.ds(off, CH)], idx_v)
            pltpu.sync_copy(eo_hbm.at[pl.ds(off, CH)], rows_v)
            pltpu.sync_copy(rows_v, shared.at[idx_v], add=True)
            return carry

        lax.fori_loop(0, NFULL, chunk, 0)
        toff = base + NFULL * CH
        pltpu.sync_copy(jj_hbm.at[pl.ds(toff, TAIL)], tidx_v)
        pltpu.sync_copy(eo_hbm.at[pl.ds(toff, TAIL)], trows_v)
        pltpu.sync_copy(trows_v, shared.at[tidx_v], add=True)

        plsc.subcore_barrier()

        @pl.when(sid < NDW)
        def _():
            pltpu.sync_copy(
                shared.at[pl.ds(sid * RPS, RPS)],
                out_hbm.at[cid, pl.ds(sid * RPS, RPS)],
            )

    return _sc_gather, _sc_scatter


BE = 4000


def _tc_edge_a_body(emb_ref, attr_ref, w1_ref, w2_ref, bq_ref):
    h = jnp.dot(emb_ref[...], w1_ref[...], preferred_element_type=jnp.float32)
    h = h * jax.nn.sigmoid(h)
    c = jnp.dot(h, w2_ref[...], preferred_element_type=jnp.float32)
    a = attr_ref[...]
    L = DO * 2 * DN
    bq = a[:, 0:1] * c[:, 0:L]
    for i in range(1, DA):
        bq += a[:, i:i + 1] * c[:, i * L:(i + 1) * L]
    bq_ref[...] = bq


def _tc_edge_a(emb, attr, w1s, w2c):
    return pl.pallas_call(
        _tc_edge_a_body,
        grid=(E // BE,),
        in_specs=[
            pl.BlockSpec((BE, EMB), lambda b: (b, 0)),
            pl.BlockSpec((BE, DA), lambda b: (b, 0)),
            pl.BlockSpec((EMB, H), lambda b: (0, 0)),
            pl.BlockSpec((H, DA * DO * 2 * DN), lambda b: (0, 0)),
        ],
        out_specs=pl.BlockSpec((BE, DO * 2 * DN), lambda b: (b, 0)),
        out_shape=jax.ShapeDtypeStruct((E, DO * 2 * DN), jnp.float32),
    )(emb, attr, w1s, w2c)


def _tc_edge_b_body(bq_ref, xi_ref, xj_ref, m_ref, out_ref, out4_ref):
    z = jnp.concatenate([xi_ref[...], xj_ref[...]], axis=1)
    z4 = jnp.concatenate([z] * DO, axis=1)
    eo = jnp.dot(bq_ref[...] * z4, m_ref[...],
                 preferred_element_type=jnp.float32)
    out_ref[...] = eo
    out4_ref[...] = eo[:, :DO]


def _tc_edge_b(bq, xi, xj, msel):
    return pl.pallas_call(
        _tc_edge_b_body,
        grid=(E // BE,),
        in_specs=[
            pl.BlockSpec((BE, DO * 2 * DN), lambda b: (b, 0)),
            pl.BlockSpec((BE, DN), lambda b: (b, 0)),
            pl.BlockSpec((BE, DN), lambda b: (b, 0)),
            pl.BlockSpec((DO * 2 * DN, DP), lambda b: (0, 0)),
        ],
        out_specs=[
            pl.BlockSpec((BE, DP), lambda b: (b, 0)),
            pl.BlockSpec((BE, DO), lambda b: (b, 0)),
        ],
        out_shape=[
            jax.ShapeDtypeStruct((E, DP), jnp.float32),
            jax.ShapeDtypeStruct((E, DO), jnp.float32),
        ],
    )(bq, xi, xj, msel)


BN = 2000


def _tc_node_body(x_ref, ps_ref, wn_ref, out_ref):
    e = (ps_ref[0] + ps_ref[1])[:, :DO]
    x = x_ref[...]
    acc = (jnp.dot(x, wn_ref[0], preferred_element_type=jnp.float32)
           * e[:, 0:1])
    for j in range(1, DO):
        acc += (jnp.dot(x, wn_ref[j], preferred_element_type=jnp.float32)
                * e[:, j:j + 1])
    out_ref[...] = acc


def _tc_node(x, ps, wn):
    return pl.pallas_call(
        _tc_node_body,
        grid=(N // BN,),
        in_specs=[
            pl.BlockSpec((BN, DN), lambda b: (b, 0)),
            pl.BlockSpec((NC, BN, DP), lambda b: (0, b, 0)),
            pl.BlockSpec((DO, DN, DN), lambda b: (0, 0, 0)),
        ],
        out_specs=pl.BlockSpec((BN, DN), lambda b: (b, 0)),
        out_shape=jax.ShapeDtypeStruct((N, DN), jnp.float32),
    )(x, ps, wn)


def kernel(x, edge_index, edge_attr, edge_len_emb, W1, W2, w_node):
    ii = edge_index[0]
    jj = edge_index[1]
    w1s = W1 * (1.0 / math.sqrt(EMB))
    w2c = (
        W2.reshape(H, DA, 2 * DN, DO).transpose(0, 1, 3, 2).reshape(H, DA * DO * 2 * DN)
        * (1.0 / (math.sqrt(H) * math.sqrt(DA * 2 * DN)))
    )
    rows = jnp.arange(DO * 2 * DN)[:, None] // (2 * DN)
    cols = jnp.arange(DP)[None, :]
    msel = (rows == cols).astype(jnp.float32)
    wn = w_node.transpose(1, 0, 2) * (1.0 / math.sqrt(DN * DO))
    zer = jnp.zeros((N, DP), jnp.float32)

    sc_gather, sc_scatter = _sc_kernels()
    xi, xj = sc_gather(x, ii, jj)
    bq = _tc_edge_a(edge_len_emb, edge_attr, w1s, w2c)
    eop, eo4 = _tc_edge_b(bq, xi, xj, msel)
    ps = sc_scatter(eop, jj, zer)
    xo = _tc_node(x, ps, wn)
    return (xo, eo4)
